# Initial kernel scaffold; baseline (speedup 1.0000x reference)
import jax
import jax.numpy as jnp
from jax.experimental import pallas as pl


def kernel(x, pos, edge_index, batch, p1, p2, lin):
    raise NotImplementedError("write your pallas kernel here")


# speedup vs baseline: 1.1604x; 1.1604x over previous
import functools
from functools import partial

import jax
import jax.numpy as jnp
from jax import lax
from jax.experimental import pallas as pl
from jax.experimental.pallas import tpu as pltpu

_BN_EPS = 1e-5
_N_NODES = 10000
_NODE_BLK = 1000
_EDGE_BLK = 2000


def _precompute_body(pos_ref, feat_ref, wa_ref, wb_ref, wc_ref, b_ref, a_out, b_out):
    pos = pos_ref[...]
    feat = feat_ref[...]
    a = jnp.dot(pos, wa_ref[...], preferred_element_type=jnp.float32)
    a = a + jnp.dot(feat, wc_ref[...], preferred_element_type=jnp.float32)
    a_out[...] = a + b_ref[...]
    b_out[...] = jnp.dot(pos, wb_ref[...], preferred_element_type=jnp.float32)


def _node_precompute(pos, feat, w1, b1):
    n, f = feat.shape
    h = w1.shape[1]
    wa = w1[0:3] - w1[3:6]
    wb = w1[3:6]
    wc = w1[6:]
    grid = n // _NODE_BLK
    return pl.pallas_call(
        _precompute_body,
        grid=(grid,),
        in_specs=[
            pl.BlockSpec((_NODE_BLK, 3), lambda i: (i, 0)),
            pl.BlockSpec((_NODE_BLK, f), lambda i: (i, 0)),
            pl.BlockSpec((3, h), lambda i: (0, 0)),
            pl.BlockSpec((3, h), lambda i: (0, 0)),
            pl.BlockSpec((f, h), lambda i: (0, 0)),
            pl.BlockSpec((1, h), lambda i: (0, 0)),
        ],
        out_specs=[
            pl.BlockSpec((_NODE_BLK, h), lambda i: (i, 0)),
            pl.BlockSpec((_NODE_BLK, h), lambda i: (i, 0)),
        ],
        out_shape=[
            jax.ShapeDtypeStruct((n, h), jnp.float32),
            jax.ShapeDtypeStruct((n, h), jnp.float32),
        ],
    )(pos, feat, wa, wb, wc, b1.reshape(1, h))


def _edge_mlp_body(ga_ref, gb_ref, w2_ref, b2_ref, w3_ref, b3_ref, s3_ref, t3_ref, out_ref):
    h1 = jnp.maximum(ga_ref[...] + gb_ref[...], 0.0)
    u2 = jnp.dot(h1, w2_ref[...], preferred_element_type=jnp.float32) + b2_ref[...]
    h2 = jnp.maximum(u2, 0.0)
    u3 = jnp.dot(h2, w3_ref[...], preferred_element_type=jnp.float32) + b3_ref[...]
    h3 = s3_ref[...] * jnp.maximum(u3, 0.0) + t3_ref[...]
    out_ref[...] = jnp.maximum(h3, 0.0)


def _edge_mlp(ga, gb, w2f, b2f, w3f, b3f, s3, t3):
    e, h = ga.shape
    grid = e // _EDGE_BLK
    return pl.pallas_call(
        _edge_mlp_body,
        grid=(grid,),
        in_specs=[
            pl.BlockSpec((_EDGE_BLK, h), lambda i: (i, 0)),
            pl.BlockSpec((_EDGE_BLK, h), lambda i: (i, 0)),
            pl.BlockSpec((h, h), lambda i: (0, 0)),
            pl.BlockSpec((1, h), lambda i: (0, 0)),
            pl.BlockSpec((h, h), lambda i: (0, 0)),
            pl.BlockSpec((1, h), lambda i: (0, 0)),
            pl.BlockSpec((1, h), lambda i: (0, 0)),
            pl.BlockSpec((1, h), lambda i: (0, 0)),
        ],
        out_specs=pl.BlockSpec((_EDGE_BLK, h), lambda i: (i, 0)),
        out_shape=jax.ShapeDtypeStruct((e, h), jnp.float32),
    )(ga, gb, w2f, b2f.reshape(1, h), w3f, b3f.reshape(1, h),
      s3.reshape(1, h), t3.reshape(1, h))


def _head_body(m1_ref, m2_ref, lw1_ref, lb1_ref, lw2_ref, lb2_ref, out_ref):
    n = m1_ref.shape[0]
    mean1 = jnp.sum(m1_ref[...], axis=0, keepdims=True) * (1.0 / n)
    mean2 = jnp.sum(m2_ref[...], axis=0, keepdims=True) * (1.0 / n)
    j = jnp.concatenate([mean1, mean2], axis=1)
    z = jnp.maximum(jnp.dot(j, lw1_ref[...], preferred_element_type=jnp.float32)
                    + lb1_ref[...], 0.0)
    logits = jnp.dot(z, lw2_ref[...], preferred_element_type=jnp.float32) + lb2_ref[...]
    mx = jnp.max(logits, axis=1, keepdims=True)
    lse = jnp.log(jnp.sum(jnp.exp(logits - mx), axis=1, keepdims=True)) + mx
    out_ref[...] = logits - lse


def _head(m1, m2, lw1, lb1, lw2, lb2):
    n, h = m1.shape
    ncls = lw2.shape[1]
    return pl.pallas_call(
        _head_body,
        out_shape=jax.ShapeDtypeStruct((1, ncls), jnp.float32),
    )(m1, m2, lw1, lb1.reshape(1, h), lw2, lb2.reshape(1, ncls))


def _fold_bn(params):
    c = 1.0 / jnp.sqrt(1.0 + _BN_EPS)
    w1, b1, g1, be1 = params[0:4]
    w2, b2, g2, be2 = params[4:8]
    w3, b3, g3, be3 = params[8:12]
    s1, t1 = g1 * c, be1
    s2, t2 = g2 * c, be2
    s3, t3 = g3 * c, be3
    w2f = s1[:, None] * w2
    b2f = t1 @ w2 + b2
    w3f = s2[:, None] * w3
    b3f = t2 @ w3 + b3
    return w1, b1, w2f, b2f, w3f, b3f, s3, t3


def _conv_layer(pos, feat, dst, src, params):
    w1, b1, w2f, b2f, w3f, b3f, s3, t3 = _fold_bn(params)
    a, b = _node_precompute(pos, feat, w1, b1)
    ga = jnp.take(a, dst, axis=0)
    gb = jnp.take(b, src, axis=0)
    r3 = _edge_mlp(ga, gb, w2f, b2f, w3f, b3f, s3, t3)
    m = jax.ops.segment_max(r3, dst, num_segments=feat.shape[0])
    return jnp.maximum(m, 0.0)


def kernel(x, pos, edge_index, batch, p1, p2, lin):
    dst = edge_index[1]
    src = edge_index[0]
    h1 = _conv_layer(pos, x, dst, src, p1)
    h2 = _conv_layer(pos, h1, dst, src, p2)
    lw1, lb1, lw2, lb2 = lin
    return _head(h1, h2, lw1, lb1, lw2, lb2)


# speedup vs baseline: 2.1672x; 1.8677x over previous
import functools
from functools import partial

import jax
import jax.numpy as jnp
from jax import lax
from jax.experimental import pallas as pl
from jax.experimental.pallas import tpu as pltpu
from jax.experimental.pallas import tpu_sc as plsc

_SC_CORES = 2
_SC_SUBCORES = 16
_SC_WORKERS = _SC_CORES * _SC_SUBCORES
_GCHUNK = 80
_GROWS = 4000
_GROWS_PW = _GROWS // _SC_WORKERS


def _sc_gather_body(a_hbm, b_hbm, dix_hbm, six_hbm, ga_hbm, gb_hbm,
                    dix_v, six_v, bufa, bufb, gsem):
    wid = lax.axis_index("s") * _SC_CORES + lax.axis_index("c")
    row0 = wid * _GROWS_PW
    pltpu.sync_copy(dix_hbm.at[wid], dix_v)
    pltpu.sync_copy(six_hbm.at[wid], six_v)

    def body(k, _):
        c1 = pltpu.async_copy(a_hbm.at[dix_v.at[k]], bufa, gsem)
        c2 = pltpu.async_copy(b_hbm.at[six_v.at[k]], bufb, gsem)
        c1.wait()
        c2.wait()
        e0 = (row0 + k) * _GCHUNK
        pltpu.sync_copy(bufa, ga_hbm.at[pl.ds(e0, _GCHUNK)])
        pltpu.sync_copy(bufb, gb_hbm.at[pl.ds(e0, _GCHUNK)])
        return 0

    lax.fori_loop(0, _GROWS_PW, body, 0)


def _sc_gather(a, b, dix2d, six2d):
    e = _GROWS * _GCHUNK
    h = a.shape[1]
    mesh = plsc.VectorSubcoreMesh(core_axis_name="c", subcore_axis_name="s")
    fn = functools.partial(
        pl.kernel,
        mesh=mesh,
        compiler_params=pltpu.CompilerParams(use_tc_tiling_on_sc=False),
        out_type=[
            jax.ShapeDtypeStruct((e, h), jnp.float32),
            jax.ShapeDtypeStruct((e, h), jnp.float32),
        ],
        scratch_types=[
            pltpu.VMEM((_GROWS_PW, _GCHUNK), jnp.int32),
            pltpu.VMEM((_GROWS_PW, _GCHUNK), jnp.int32),
            pltpu.VMEM((_GCHUNK, h), jnp.float32),
            pltpu.VMEM((_GCHUNK, h), jnp.float32),
            pltpu.SemaphoreType.DMA,
        ],
    )(_sc_gather_body)
    return fn(a, b, dix2d, six2d)

_BN_EPS = 1e-5
_N_NODES = 10000
_NODE_BLK = 1000
_EDGE_BLK = 2000


def _precompute_body(pos_ref, feat_ref, wa_ref, wb_ref, wc_ref, b_ref, a_out, b_out):
    pos = pos_ref[...]
    feat = feat_ref[...]
    a = jnp.dot(pos, wa_ref[...], preferred_element_type=jnp.float32)
    a = a + jnp.dot(feat, wc_ref[...], preferred_element_type=jnp.float32)
    a_out[...] = a + b_ref[...]
    b_out[...] = jnp.dot(pos, wb_ref[...], preferred_element_type=jnp.float32)


def _node_precompute(pos, feat, w1, b1):
    n, f = feat.shape
    h = w1.shape[1]
    wa = w1[0:3] - w1[3:6]
    wb = w1[3:6]
    wc = w1[6:]
    grid = n // _NODE_BLK
    return pl.pallas_call(
        _precompute_body,
        grid=(grid,),
        in_specs=[
            pl.BlockSpec((_NODE_BLK, 3), lambda i: (i, 0)),
            pl.BlockSpec((_NODE_BLK, f), lambda i: (i, 0)),
            pl.BlockSpec((3, h), lambda i: (0, 0)),
            pl.BlockSpec((3, h), lambda i: (0, 0)),
            pl.BlockSpec((f, h), lambda i: (0, 0)),
            pl.BlockSpec((1, h), lambda i: (0, 0)),
        ],
        out_specs=[
            pl.BlockSpec((_NODE_BLK, h), lambda i: (i, 0)),
            pl.BlockSpec((_NODE_BLK, h), lambda i: (i, 0)),
        ],
        out_shape=[
            jax.ShapeDtypeStruct((n, h), jnp.float32),
            jax.ShapeDtypeStruct((n, h), jnp.float32),
        ],
    )(pos, feat, wa, wb, wc, b1.reshape(1, h))


def _edge_mlp_body(ga_ref, gb_ref, w2_ref, b2_ref, w3_ref, b3_ref, s3_ref, t3_ref, out_ref):
    h1 = jnp.maximum(ga_ref[...] + gb_ref[...], 0.0)
    u2 = jnp.dot(h1, w2_ref[...], preferred_element_type=jnp.float32) + b2_ref[...]
    h2 = jnp.maximum(u2, 0.0)
    u3 = jnp.dot(h2, w3_ref[...], preferred_element_type=jnp.float32) + b3_ref[...]
    h3 = s3_ref[...] * jnp.maximum(u3, 0.0) + t3_ref[...]
    out_ref[...] = jnp.maximum(h3, 0.0)


def _edge_mlp(ga, gb, w2f, b2f, w3f, b3f, s3, t3):
    e, h = ga.shape
    grid = e // _EDGE_BLK
    return pl.pallas_call(
        _edge_mlp_body,
        grid=(grid,),
        in_specs=[
            pl.BlockSpec((_EDGE_BLK, h), lambda i: (i, 0)),
            pl.BlockSpec((_EDGE_BLK, h), lambda i: (i, 0)),
            pl.BlockSpec((h, h), lambda i: (0, 0)),
            pl.BlockSpec((1, h), lambda i: (0, 0)),
            pl.BlockSpec((h, h), lambda i: (0, 0)),
            pl.BlockSpec((1, h), lambda i: (0, 0)),
            pl.BlockSpec((1, h), lambda i: (0, 0)),
            pl.BlockSpec((1, h), lambda i: (0, 0)),
        ],
        out_specs=pl.BlockSpec((_EDGE_BLK, h), lambda i: (i, 0)),
        out_shape=jax.ShapeDtypeStruct((e, h), jnp.float32),
    )(ga, gb, w2f, b2f.reshape(1, h), w3f, b3f.reshape(1, h),
      s3.reshape(1, h), t3.reshape(1, h))


def _head_body(m1_ref, m2_ref, lw1_ref, lb1_ref, lw2_ref, lb2_ref, out_ref):
    n = m1_ref.shape[0]
    mean1 = jnp.sum(m1_ref[...], axis=0, keepdims=True) * (1.0 / n)
    mean2 = jnp.sum(m2_ref[...], axis=0, keepdims=True) * (1.0 / n)
    j = jnp.concatenate([mean1, mean2], axis=1)
    z = jnp.maximum(jnp.dot(j, lw1_ref[...], preferred_element_type=jnp.float32)
                    + lb1_ref[...], 0.0)
    logits = jnp.dot(z, lw2_ref[...], preferred_element_type=jnp.float32) + lb2_ref[...]
    mx = jnp.max(logits, axis=1, keepdims=True)
    lse = jnp.log(jnp.sum(jnp.exp(logits - mx), axis=1, keepdims=True)) + mx
    out_ref[...] = logits - lse


def _head(m1, m2, lw1, lb1, lw2, lb2):
    n, h = m1.shape
    ncls = lw2.shape[1]
    return pl.pallas_call(
        _head_body,
        out_shape=jax.ShapeDtypeStruct((1, ncls), jnp.float32),
    )(m1, m2, lw1, lb1.reshape(1, h), lw2, lb2.reshape(1, ncls))


def _fold_bn(params):
    c = 1.0 / jnp.sqrt(1.0 + _BN_EPS)
    w1, b1, g1, be1 = params[0:4]
    w2, b2, g2, be2 = params[4:8]
    w3, b3, g3, be3 = params[8:12]
    s1, t1 = g1 * c, be1
    s2, t2 = g2 * c, be2
    s3, t3 = g3 * c, be3
    w2f = s1[:, None] * w2
    b2f = t1 @ w2 + b2
    w3f = s2[:, None] * w3
    b3f = t2 @ w3 + b3
    return w1, b1, w2f, b2f, w3f, b3f, s3, t3


def _conv_layer(pos, feat, dst, src, dix2d, six2d, params):
    w1, b1, w2f, b2f, w3f, b3f, s3, t3 = _fold_bn(params)
    a, b = _node_precompute(pos, feat, w1, b1)
    ga, gb = _sc_gather(a, b, dix2d, six2d)
    r3 = _edge_mlp(ga, gb, w2f, b2f, w3f, b3f, s3, t3)
    m = jax.ops.segment_max(r3, dst, num_segments=feat.shape[0])
    return jnp.maximum(m, 0.0)


def kernel(x, pos, edge_index, batch, p1, p2, lin):
    dst = edge_index[1]
    src = edge_index[0]
    dix2d = dst.reshape(_SC_WORKERS, _GROWS_PW, _GCHUNK)
    six2d = src.reshape(_SC_WORKERS, _GROWS_PW, _GCHUNK)
    h1 = _conv_layer(pos, x, dst, src, dix2d, six2d, p1)
    h2 = _conv_layer(pos, h1, dst, src, dix2d, six2d, p2)
    lw1, lb1, lw2, lb2 = lin
    return _head(h1, h2, lw1, lb1, lw2, lb2)
